# Initial kernel scaffold; baseline (speedup 1.0000x reference)
import jax
import jax.numpy as jnp
from jax.experimental import pallas as pl


def kernel(x0, x1, convert_matrix):
    raise NotImplementedError("write your pallas kernel here")


# speedup vs baseline: 745.9507x; 745.9507x over previous
import jax
import jax.numpy as jnp
from jax.experimental import pallas as pl
from jax.experimental.pallas import tpu as pltpu

B = 2
N = 20000
NC = 80
NM = 32
PH = 160
PW = 160
MAX_OBJ = 100
SCORE_THRES = 0.25
MASK_RES = 56
POOLER_SCALE = 0.25

NPAD = 20480
NROW = 160
NCHUNK = 16
T = B * MAX_OBJ
ROW_W = 5 + NC + NM
NEG = -1e30


def _k1_scores(x_ref, nms_ref):
    x = x_ref[0]
    conf = x[:, :, 4:5]
    prod = x * conf
    lane = jax.lax.broadcasted_iota(jnp.int32, (NCHUNK, 128, ROW_W), 2)
    prod = jnp.where((lane >= 5) & (lane < 5 + NC), prod, NEG)
    mx = jnp.max(prod, axis=2)
    g = pl.program_id(1)
    r = jax.lax.broadcasted_iota(jnp.int32, (NCHUNK, 128), 0) + g * NCHUNK
    c = jax.lax.broadcasted_iota(jnp.int32, (NCHUNK, 128), 1)
    n = r * 128 + c
    keep = (n < N) & (mx > SCORE_THRES)
    nms_ref[0] = jnp.where(keep, mx, NEG)


def _k2_topk(nms_ref, idx_ref):
    v0 = nms_ref[0]
    r = jax.lax.broadcasted_iota(jnp.int32, (NROW, 128), 0)
    c = jax.lax.broadcasted_iota(jnp.int32, (NROW, 128), 1)
    n = r * 128 + c
    lane = jax.lax.broadcasted_iota(jnp.int32, (1, 128), 1)

    def body(k, carry):
        v, acc = carry
        m = jnp.max(v)
        idx = jnp.min(jnp.where(v == m, n, jnp.int32(2 ** 30)))
        v = jnp.where(n == idx, -jnp.inf, v)
        acc = jnp.where(lane == k, idx, acc)
        return v, acc

    acc0 = jnp.zeros((1, 128), jnp.int32)
    _, acc = jax.lax.fori_loop(0, MAX_OBJ, body, (v0, acc0))
    idx_ref[0] = acc


def _k3_gather(y_ref, x_ref, cm_ref, sel_ref):
    t = pl.program_id(0)
    row = x_ref[0, 0]
    conf = row[0, 4]
    prod = row[:, 5:5 + NC] * conf
    mx = jnp.max(prod)
    lane = jax.lax.broadcasted_iota(jnp.int32, (1, NC), 1)
    cls_i = jnp.min(jnp.where(prod == mx, lane, jnp.int32(1000)))
    cm = cm_ref[...]
    box = (row[0, 0] * cm[0:1, :] + row[0, 1] * cm[1:2, :]
           + row[0, 2] * cm[2:3, :] + row[0, 3] * cm[3:4, :])
    x_b = t // MAX_OBJ
    s_val = (x_b + cls_i + y_ref[t]).astype(jnp.float32)
    out = jnp.concatenate(
        [box,
         jnp.full((1, 1), mx, jnp.float32),
         jnp.full((1, 1), cls_i.astype(jnp.float32), jnp.float32),
         row[:, 5 + NC:],
         jnp.full((1, 1), s_val, jnp.float32)], axis=1)
    sel_ref[0] = out


def _k4_finalize(sel_ref, numdet_ref, dets_ref, m64_ref):
    sel = sel_ref[...]
    s = sel[:, 38:39]
    ti = jax.lax.broadcasted_iota(jnp.int32, (T, 1), 0)
    no1 = jnp.max(jnp.where(s > 0.0, ti, 0)) + 1
    lag = s[1:, :] - s[:-1, :]
    ti1 = jax.lax.broadcasted_iota(jnp.int32, (T - 1, 1), 0)
    no2 = jnp.max(jnp.where(lag != 0.0, ti1, 0)) + 2
    cond = sel[T - 1, 38] != sel[0, 38]
    num_obj = jnp.where(cond, jnp.minimum(no1, no2), 0)
    n0 = jnp.clip(num_obj, 0, MAX_OBJ)
    n1 = jnp.clip(num_obj - MAX_OBJ, 0, MAX_OBJ)
    bi = jax.lax.broadcasted_iota(jnp.int32, (B, 1), 0)
    numdet_ref[...] = jnp.where(bi == 0, n0, n1)
    idxs = jnp.where(ti < MAX_OBJ,
                     jnp.maximum(n0 - 1 - ti, 0),
                     jnp.where(ti - MAX_OBJ < n1, (T - 1) + n1 - ti, 0))
    lane = jax.lax.broadcasted_iota(jnp.int32, (T, T), 1)
    oh = (lane == idxs).astype(jnp.float32)
    g = jnp.dot(oh, sel[:, 0:38], preferred_element_type=jnp.float32)
    dets_ref[...] = g[:, 0:6]
    mskp = g[:, 6:38]
    b = idxs // MAX_OBJ
    m64_ref[...] = jnp.concatenate(
        [jnp.where(b == 0, mskp, 0.0), jnp.where(b == 1, mskp, 0.0)], axis=1)


def _k5_bigmm(m_ref, p_ref, w_ref):
    w_ref[...] = jnp.dot(m_ref[...], p_ref[...],
                         preferred_element_type=jnp.float32)


def _k6_roialign(box_ref, w_ref, out_ref):
    f32 = jnp.float32
    bx1 = box_ref[0, 0, 0] * POOLER_SCALE - 0.5
    by1 = box_ref[0, 0, 1] * POOLER_SCALE - 0.5
    bx2 = box_ref[0, 0, 2] * POOLER_SCALE - 0.5
    by2 = box_ref[0, 0, 3] * POOLER_SCALE - 0.5
    bh = (by2 - by1) / f32(MASK_RES)
    bw = (bx2 - bx1) / f32(MASK_RES)

    rP = jax.lax.broadcasted_iota(jnp.int32, (MASK_RES, PH), 0).astype(f32)
    jP = jax.lax.broadcasted_iota(jnp.int32, (MASK_RES, PH), 1).astype(f32)

    def wy(i_f):
        y = by1 + (i_f + 0.5) * bh * 0.5
        y = jnp.clip(y, 0.0, f32(PH - 1))
        y0 = jnp.clip(jnp.floor(y), 0.0, f32(PH - 2))
        ly = y - y0
        return (jnp.where(jP == y0, 1.0 - ly, 0.0)
                + jnp.where(jP == y0 + 1.0, ly, 0.0))

    P = 0.5 * (wy(2.0 * rP) + wy(2.0 * rP + 1.0))

    jQ = jax.lax.broadcasted_iota(jnp.int32, (PW, MASK_RES), 0).astype(f32)
    cQ = jax.lax.broadcasted_iota(jnp.int32, (PW, MASK_RES), 1).astype(f32)

    def wx(i_f):
        x = bx1 + (i_f + 0.5) * bw * 0.5
        x = jnp.clip(x, 0.0, f32(PW - 1))
        x0 = jnp.clip(jnp.floor(x), 0.0, f32(PW - 2))
        lx = x - x0
        return (jnp.where(jQ == x0, 1.0 - lx, 0.0)
                + jnp.where(jQ == x0 + 1.0, lx, 0.0))

    Q = 0.5 * (wx(2.0 * cQ) + wx(2.0 * cQ + 1.0))

    S = jnp.dot(P, w_ref[0], preferred_element_type=f32)
    S = jnp.dot(S, Q, preferred_element_type=f32)
    out_ref[0] = jax.nn.sigmoid(S)


def kernel(x0, x1, convert_matrix):
    f32 = jnp.float32
    x0p = jnp.pad(x0, ((0, 0), (0, NPAD - N), (0, 0)))
    x4 = x0p.reshape(B, NROW, 128, ROW_W)
    nms = pl.pallas_call(
        _k1_scores,
        grid=(B, NROW // NCHUNK),
        in_specs=[pl.BlockSpec((1, NCHUNK, 128, ROW_W),
                               lambda b, g: (b, g, 0, 0))],
        out_specs=pl.BlockSpec((1, NCHUNK, 128), lambda b, g: (b, g, 0)),
        out_shape=jax.ShapeDtypeStruct((B, NROW, 128), f32),
    )(x4)

    topk = pl.pallas_call(
        _k2_topk,
        grid=(B,),
        in_specs=[pl.BlockSpec((1, NROW, 128), lambda b: (b, 0, 0))],
        out_specs=pl.BlockSpec((1, 1, 128), lambda b: (b, 0, 0)),
        out_shape=jax.ShapeDtypeStruct((B, 1, 128), jnp.int32),
    )(nms)
    y_idx = topk.reshape(B, 128)[:, :MAX_OBJ].reshape(T)

    x0r = x0.reshape(B, N, 1, ROW_W)
    sel = pl.pallas_call(
        _k3_gather,
        grid_spec=pltpu.PrefetchScalarGridSpec(
            num_scalar_prefetch=1,
            grid=(T,),
            in_specs=[
                pl.BlockSpec((1, 1, 1, ROW_W),
                             lambda t, y: (t // MAX_OBJ, y[t], 0, 0)),
                pl.BlockSpec((4, 4), lambda t, y: (0, 0)),
            ],
            out_specs=pl.BlockSpec((1, 1, 39), lambda t, y: (t, 0, 0)),
        ),
        out_shape=jax.ShapeDtypeStruct((T, 1, 39), f32),
    )(y_idx, x0r, convert_matrix)
    sel2 = sel.reshape(T, 39)

    num_det, dets, m64 = pl.pallas_call(
        _k4_finalize,
        out_shape=[
            jax.ShapeDtypeStruct((B, 1), jnp.int32),
            jax.ShapeDtypeStruct((T, 6), f32),
            jax.ShapeDtypeStruct((T, 2 * NM), f32),
        ],
    )(sel2)

    proto2 = x1.reshape(B * NM, PH * PW)
    CB = 3200
    weighted = pl.pallas_call(
        _k5_bigmm,
        grid=(PH * PW // CB,),
        in_specs=[
            pl.BlockSpec((T, 2 * NM), lambda j: (0, 0)),
            pl.BlockSpec((2 * NM, CB), lambda j: (0, j)),
        ],
        out_specs=pl.BlockSpec((T, CB), lambda j: (0, j)),
        out_shape=jax.ShapeDtypeStruct((T, PH * PW), f32),
    )(m64, proto2)
    w3 = weighted.reshape(T, PH, PW)

    det_box3 = dets[:, 0:4].reshape(T, 1, 4)
    masks = pl.pallas_call(
        _k6_roialign,
        grid=(T,),
        in_specs=[
            pl.BlockSpec((1, 1, 4), lambda t: (t, 0, 0)),
            pl.BlockSpec((1, PH, PW), lambda t: (t, 0, 0)),
        ],
        out_specs=pl.BlockSpec((1, MASK_RES, MASK_RES), lambda t: (t, 0, 0)),
        out_shape=jax.ShapeDtypeStruct((T, MASK_RES, MASK_RES), f32),
    )(det_box3, w3)

    det_boxes = dets[:, 0:4].reshape(B, MAX_OBJ, 4)
    det_scores = dets[:, 4:5].reshape(B, MAX_OBJ, 1)
    det_classes = dets[:, 5:6].reshape(B, MAX_OBJ, 1)
    det_masks = masks.reshape(B, MAX_OBJ, MASK_RES * MASK_RES)
    return (num_det, det_boxes, det_scores, det_classes, det_masks)


# speedup vs baseline: 1431.1899x; 1.9186x over previous
import jax
import jax.numpy as jnp
from jax.experimental import pallas as pl
from jax.experimental.pallas import tpu as pltpu

B = 2
N = 20000
NC = 80
NM = 32
PH = 160
PW = 160
MAX_OBJ = 100
SCORE_THRES = 0.25
MASK_RES = 56
POOLER_SCALE = 0.25

NPAD = 20480
NROW = 160
NCHUNK = 16
T = B * MAX_OBJ
ROW_W = 5 + NC + NM
NEG = -1e30


K1_ROWS = 512


def _k1_scores(x_ref, nms_ref):
    x = x_ref[0]
    conf = x[:, 4:5]
    prod = x * conf
    lane = jax.lax.broadcasted_iota(jnp.int32, (K1_ROWS, ROW_W), 1)
    prod = jnp.where((lane >= 5) & (lane < 5 + NC), prod, NEG)
    mx = jnp.max(prod, axis=1, keepdims=True)
    g = pl.program_id(1)
    s = jax.lax.broadcasted_iota(jnp.int32, (K1_ROWS, 1), 0)
    n = g * K1_ROWS + s
    keep = (n < N) & (mx > SCORE_THRES)
    out = jnp.where(keep, mx, NEG)
    nms_ref[0, 0] = out.reshape(K1_ROWS // 128, 128)


def _k2_topk(nms_ref, idx_ref):
    v0 = nms_ref[0].reshape(NROW, 128)
    r = jax.lax.broadcasted_iota(jnp.int32, (NROW, 128), 0)
    c = jax.lax.broadcasted_iota(jnp.int32, (NROW, 128), 1)
    n = r * 128 + c
    lane = jax.lax.broadcasted_iota(jnp.int32, (1, 128), 1)

    def body(k, carry):
        v, acc = carry
        m = jnp.max(v)
        idx = jnp.min(jnp.where(v == m, n, jnp.int32(2 ** 30)))
        v = jnp.where(n == idx, -jnp.inf, v)
        acc = jnp.where(lane == k, idx, acc)
        return v, acc

    acc0 = jnp.zeros((1, 128), jnp.int32)
    _, acc = jax.lax.fori_loop(0, MAX_OBJ, body, (v0, acc0))
    idx_ref[0] = acc


K3_R = 8


def _k3_gather(y_ref, *refs):
    t = pl.program_id(0)
    row_refs = refs[:K3_R]
    cm_ref, sel_ref = refs[K3_R], refs[K3_R + 1]
    rows = jnp.concatenate([r[0, 0] for r in row_refs], axis=0)
    conf = rows[:, 4:5]
    prod = rows[:, 5:5 + NC] * conf
    mx = jnp.max(prod, axis=1, keepdims=True)
    lane = jax.lax.broadcasted_iota(jnp.int32, (K3_R, NC), 1)
    cls_i = jnp.min(jnp.where(prod == mx, lane, jnp.int32(1000)),
                    axis=1, keepdims=True)
    cm = cm_ref[...]
    box = (rows[:, 0:1] * cm[0:1, :] + rows[:, 1:2] * cm[1:2, :]
           + rows[:, 2:3] * cm[2:3, :] + rows[:, 3:4] * cm[3:4, :])
    tv = t * K3_R + jax.lax.broadcasted_iota(jnp.int32, (K3_R, 1), 0)
    x_b = tv // MAX_OBJ
    yv = jnp.concatenate(
        [jnp.full((1, 1), y_ref[t * K3_R + j], jnp.int32)
         for j in range(K3_R)], axis=0)
    s_val = (x_b + cls_i + yv).astype(jnp.float32)
    out = jnp.concatenate(
        [box, mx, cls_i.astype(jnp.float32), rows[:, 5 + NC:], s_val],
        axis=1)
    sel_ref[0] = out


def _k4_finalize(sel_ref, numdet_ref, dets_ref, m64_ref):
    sel = sel_ref[...]
    s = sel[:, 38:39]
    ti = jax.lax.broadcasted_iota(jnp.int32, (T, 1), 0)
    no1 = jnp.max(jnp.where(s > 0.0, ti, 0)) + 1
    lag = s[1:, :] - s[:-1, :]
    ti1 = jax.lax.broadcasted_iota(jnp.int32, (T - 1, 1), 0)
    no2 = jnp.max(jnp.where(lag != 0.0, ti1, 0)) + 2
    cond = sel[T - 1, 38] != sel[0, 38]
    num_obj = jnp.where(cond, jnp.minimum(no1, no2), 0)
    n0 = jnp.clip(num_obj, 0, MAX_OBJ)
    n1 = jnp.clip(num_obj - MAX_OBJ, 0, MAX_OBJ)
    bi = jax.lax.broadcasted_iota(jnp.int32, (B, 1), 0)
    numdet_ref[...] = jnp.where(bi == 0, n0, n1)
    idxs = jnp.where(ti < MAX_OBJ,
                     jnp.maximum(n0 - 1 - ti, 0),
                     jnp.where(ti - MAX_OBJ < n1, (T - 1) + n1 - ti, 0))
    lane = jax.lax.broadcasted_iota(jnp.int32, (T, T), 1)
    oh = (lane == idxs).astype(jnp.float32)
    g = jnp.dot(oh, sel[:, 0:38], preferred_element_type=jnp.float32)
    dets_ref[...] = g[:, 0:6]
    mskp = g[:, 6:38]
    b = idxs // MAX_OBJ
    m64_ref[...] = jnp.concatenate(
        [jnp.where(b == 0, mskp, 0.0), jnp.where(b == 1, mskp, 0.0)], axis=1)


def _k5_bigmm(m_ref, p_ref, w_ref):
    w_ref[...] = jnp.dot(m_ref[...], p_ref[...],
                         preferred_element_type=jnp.float32)


K6_R = 4


def _k6_roialign(box_ref, w_ref, out_ref):
    f32 = jnp.float32
    rP = jax.lax.broadcasted_iota(jnp.int32, (MASK_RES, PH), 0).astype(f32)
    jP = jax.lax.broadcasted_iota(jnp.int32, (MASK_RES, PH), 1).astype(f32)
    jQ = jax.lax.broadcasted_iota(jnp.int32, (PW, MASK_RES), 0).astype(f32)
    cQ = jax.lax.broadcasted_iota(jnp.int32, (PW, MASK_RES), 1).astype(f32)

    for j in range(K6_R):
        bx1 = box_ref[0, j, 0] * POOLER_SCALE - 0.5
        by1 = box_ref[0, j, 1] * POOLER_SCALE - 0.5
        bx2 = box_ref[0, j, 2] * POOLER_SCALE - 0.5
        by2 = box_ref[0, j, 3] * POOLER_SCALE - 0.5
        bh = (by2 - by1) / f32(MASK_RES)
        bw = (bx2 - bx1) / f32(MASK_RES)

        def wy(i_f):
            y = jnp.clip(by1 + (i_f + 0.5) * bh * 0.5, 0.0, f32(PH - 1))
            return jnp.maximum(0.0, 1.0 - jnp.abs(y - jP))

        P = 0.5 * (wy(2.0 * rP) + wy(2.0 * rP + 1.0))

        def wx(i_f):
            x = jnp.clip(bx1 + (i_f + 0.5) * bw * 0.5, 0.0, f32(PW - 1))
            return jnp.maximum(0.0, 1.0 - jnp.abs(x - jQ))

        Q = 0.5 * (wx(2.0 * cQ) + wx(2.0 * cQ + 1.0))

        S = jnp.dot(P, w_ref[0, j], preferred_element_type=f32)
        S = jnp.dot(S, Q, preferred_element_type=f32)
        out_ref[0, j] = jax.nn.sigmoid(S)


def kernel(x0, x1, convert_matrix):
    f32 = jnp.float32
    NB1 = NPAD // K1_ROWS
    SB1 = K1_ROWS // 128
    nms = pl.pallas_call(
        _k1_scores,
        grid=(B, NB1),
        in_specs=[pl.BlockSpec((1, K1_ROWS, ROW_W), lambda b, g: (b, g, 0))],
        out_specs=pl.BlockSpec((1, 1, SB1, 128), lambda b, g: (b, g, 0, 0)),
        out_shape=jax.ShapeDtypeStruct((B, NB1, SB1, 128), f32),
    )(x0)

    topk = pl.pallas_call(
        _k2_topk,
        grid=(B,),
        in_specs=[pl.BlockSpec((1, NB1, SB1, 128), lambda b: (b, 0, 0, 0))],
        out_specs=pl.BlockSpec((1, 1, 128), lambda b: (b, 0, 0)),
        out_shape=jax.ShapeDtypeStruct((B, 1, 128), jnp.int32),
    )(nms)
    y_idx = topk.reshape(B, 128)[:, :MAX_OBJ].reshape(T)

    x0r = x0.reshape(B, N, 1, ROW_W)
    row_specs = [
        pl.BlockSpec((1, 1, 1, ROW_W),
                     lambda t, y, j=j: ((t * K3_R + j) // MAX_OBJ,
                                        y[t * K3_R + j], 0, 0))
        for j in range(K3_R)
    ]
    sel = pl.pallas_call(
        _k3_gather,
        grid_spec=pltpu.PrefetchScalarGridSpec(
            num_scalar_prefetch=1,
            grid=(T // K3_R,),
            in_specs=row_specs + [pl.BlockSpec((4, 4), lambda t, y: (0, 0))],
            out_specs=pl.BlockSpec((1, K3_R, 39), lambda t, y: (t, 0, 0)),
        ),
        out_shape=jax.ShapeDtypeStruct((T // K3_R, K3_R, 39), f32),
    )(y_idx, *([x0r] * K3_R), convert_matrix)
    sel2 = sel.reshape(T, 39)

    num_det, dets, m64 = pl.pallas_call(
        _k4_finalize,
        out_shape=[
            jax.ShapeDtypeStruct((B, 1), jnp.int32),
            jax.ShapeDtypeStruct((T, 6), f32),
            jax.ShapeDtypeStruct((T, 2 * NM), f32),
        ],
    )(sel2)

    proto2 = x1.reshape(B * NM, PH * PW)
    CB = 3200
    weighted = pl.pallas_call(
        _k5_bigmm,
        grid=(PH * PW // CB,),
        in_specs=[
            pl.BlockSpec((T, 2 * NM), lambda j: (0, 0)),
            pl.BlockSpec((2 * NM, CB), lambda j: (0, j)),
        ],
        out_specs=pl.BlockSpec((T, CB), lambda j: (0, j)),
        out_shape=jax.ShapeDtypeStruct((T, PH * PW), f32),
    )(m64, proto2)
    w3 = weighted.reshape(T, PH, PW)

    NB6 = T // K6_R
    det_box3 = dets[:, 0:4].reshape(NB6, K6_R, 4)
    w4 = w3.reshape(NB6, K6_R, PH, PW)
    masks = pl.pallas_call(
        _k6_roialign,
        grid=(NB6,),
        in_specs=[
            pl.BlockSpec((1, K6_R, 4), lambda t: (t, 0, 0)),
            pl.BlockSpec((1, K6_R, PH, PW), lambda t: (t, 0, 0, 0)),
        ],
        out_specs=pl.BlockSpec((1, K6_R, MASK_RES, MASK_RES),
                               lambda t: (t, 0, 0, 0)),
        out_shape=jax.ShapeDtypeStruct((NB6, K6_R, MASK_RES, MASK_RES), f32),
    )(det_box3, w4)

    det_boxes = dets[:, 0:4].reshape(B, MAX_OBJ, 4)
    det_scores = dets[:, 4:5].reshape(B, MAX_OBJ, 1)
    det_classes = dets[:, 5:6].reshape(B, MAX_OBJ, 1)
    det_masks = masks.reshape(B, MAX_OBJ, MASK_RES * MASK_RES)
    return (num_det, det_boxes, det_scores, det_classes, det_masks)
